# baseline (speedup 1.0000x reference)
import jax
import jax.numpy as jnp
from jax.experimental import pallas as pl
from jax.experimental.pallas import tpu as pltpu

_CHUNK = 4000
_NBUF = 8


def _mlp_body(x_hbm, w1_ref, b1_ref, w2_ref, b2_ref, o_hbm,
              x_buf, o_buf, in_sems, out_sems):
    n = x_hbm.shape[0]
    nchunks = n // _CHUNK

    def in_copy(i, slot):
        return pltpu.make_async_copy(
            x_hbm.at[pl.ds(i * _CHUNK, _CHUNK), :],
            x_buf.at[slot],
            in_sems.at[slot],
        )

    def out_copy(i, slot):
        return pltpu.make_async_copy(
            o_buf.at[slot],
            o_hbm.at[pl.ds(i * _CHUNK, _CHUNK), :],
            out_sems.at[slot],
        )

    for k in range(_NBUF):
        in_copy(k, k).start()

    def mlp(xblk):
        h = jnp.tanh(
            jnp.dot(
                xblk.astype(jnp.bfloat16),
                w1_ref[...],
                preferred_element_type=jnp.float32,
            )
            + b1_ref[...]
        )
        return jnp.tanh(
            jnp.dot(
                h.astype(jnp.bfloat16),
                w2_ref[...],
                preferred_element_type=jnp.float32,
            )
            + b2_ref[...]
        )

    def step(i, carry):
        slot = jax.lax.rem(i, _NBUF)

        @pl.when(i >= _NBUF)
        def _():
            out_copy(i - _NBUF, slot).wait()

        in_copy(i, slot).wait()
        o_buf[slot] = mlp(x_buf[slot])
        out_copy(i, slot).start()

        @pl.when(i + _NBUF < nchunks)
        def _():
            in_copy(i + _NBUF, slot).start()

        return carry

    jax.lax.fori_loop(0, nchunks - 1, step, 0)

    last = nchunks - 1
    lslot = last % _NBUF
    out_copy(last - _NBUF, lslot).wait()
    in_copy(last, lslot).wait()
    quarter = _CHUNK // 4
    for q in range(4):
        rows = pl.ds(q * quarter, quarter)
        o_buf[lslot, rows, :] = mlp(x_buf[lslot, rows, :])
        pltpu.make_async_copy(
            o_buf.at[lslot, rows, :],
            o_hbm.at[pl.ds(last * _CHUNK + q * quarter, quarter), :],
            out_sems.at[lslot],
        ).start()
    for q in range(4):
        pltpu.make_async_copy(
            o_buf.at[lslot, pl.ds(q * quarter, quarter), :],
            o_hbm.at[pl.ds(last * _CHUNK + q * quarter, quarter), :],
            out_sems.at[lslot],
        ).wait()

    for k in range(nchunks - _NBUF, nchunks - 1):
        out_copy(k, k % _NBUF).wait()


def kernel(x, W1, b1, W2, b2):
    n, hid = x.shape
    return pl.pallas_call(
        _mlp_body,
        in_specs=[
            pl.BlockSpec(memory_space=pl.ANY),
            pl.BlockSpec(memory_space=pltpu.MemorySpace.VMEM),
            pl.BlockSpec(memory_space=pltpu.MemorySpace.VMEM),
            pl.BlockSpec(memory_space=pltpu.MemorySpace.VMEM),
            pl.BlockSpec(memory_space=pltpu.MemorySpace.VMEM),
        ],
        out_specs=pl.BlockSpec(memory_space=pl.ANY),
        out_shape=jax.ShapeDtypeStruct((n, hid), jnp.float32),
        scratch_shapes=[
            pltpu.VMEM((_NBUF, _CHUNK, hid), jnp.float32),
            pltpu.VMEM((_NBUF, _CHUNK, hid), jnp.float32),
            pltpu.SemaphoreType.DMA((_NBUF,)),
            pltpu.SemaphoreType.DMA((_NBUF,)),
        ],
        compiler_params=pltpu.CompilerParams(
            disable_semaphore_checks=True,
        ),
    )(
        x,
        W1.T.astype(jnp.bfloat16),
        b1.reshape(1, hid),
        W2.T.astype(jnp.bfloat16),
        b2.reshape(1, hid),
    )


# speedup vs baseline: 1.0022x; 1.0022x over previous
import jax
import jax.numpy as jnp
from jax.experimental import pallas as pl
from jax.experimental.pallas import tpu as pltpu

_CHUNK = 4000
_NBUF = 12


def _mlp_body(x_hbm, w1_ref, b1_ref, w2_ref, b2_ref, o_hbm,
              x_buf, o_buf, in_sems, out_sems):
    n = x_hbm.shape[0]
    nchunks = n // _CHUNK

    def in_copy(i, slot):
        return pltpu.make_async_copy(
            x_hbm.at[pl.ds(i * _CHUNK, _CHUNK), :],
            x_buf.at[slot],
            in_sems.at[slot],
        )

    def out_copy(i, slot):
        return pltpu.make_async_copy(
            o_buf.at[slot],
            o_hbm.at[pl.ds(i * _CHUNK, _CHUNK), :],
            out_sems.at[slot],
        )

    for k in range(_NBUF):
        in_copy(k, k).start()

    def mlp(xblk):
        h = jnp.tanh(
            jnp.dot(
                xblk.astype(jnp.bfloat16),
                w1_ref[...],
                preferred_element_type=jnp.float32,
            )
            + b1_ref[...]
        )
        return jnp.tanh(
            jnp.dot(
                h.astype(jnp.bfloat16),
                w2_ref[...],
                preferred_element_type=jnp.float32,
            )
            + b2_ref[...]
        )

    def step(i, carry):
        slot = jax.lax.rem(i, _NBUF)

        @pl.when(i >= _NBUF)
        def _():
            out_copy(i - _NBUF, slot).wait()

        in_copy(i, slot).wait()
        o_buf[slot] = mlp(x_buf[slot])
        out_copy(i, slot).start()

        @pl.when(i + _NBUF < nchunks)
        def _():
            in_copy(i + _NBUF, slot).start()

        return carry

    jax.lax.fori_loop(0, nchunks, step, 0)

    for k in range(nchunks - _NBUF, nchunks):
        out_copy(k, k % _NBUF).wait()


def kernel(x, W1, b1, W2, b2):
    n, hid = x.shape
    return pl.pallas_call(
        _mlp_body,
        in_specs=[
            pl.BlockSpec(memory_space=pl.ANY),
            pl.BlockSpec(memory_space=pltpu.MemorySpace.VMEM),
            pl.BlockSpec(memory_space=pltpu.MemorySpace.VMEM),
            pl.BlockSpec(memory_space=pltpu.MemorySpace.VMEM),
            pl.BlockSpec(memory_space=pltpu.MemorySpace.VMEM),
        ],
        out_specs=pl.BlockSpec(memory_space=pl.ANY),
        out_shape=jax.ShapeDtypeStruct((n, hid), jnp.float32),
        scratch_shapes=[
            pltpu.VMEM((_NBUF, _CHUNK, hid), jnp.float32),
            pltpu.VMEM((_NBUF, _CHUNK, hid), jnp.float32),
            pltpu.SemaphoreType.DMA((_NBUF,)),
            pltpu.SemaphoreType.DMA((_NBUF,)),
        ],
        compiler_params=pltpu.CompilerParams(
            disable_semaphore_checks=True,
        ),
    )(
        x,
        W1.T.astype(jnp.bfloat16),
        b1.reshape(1, hid),
        W2.T.astype(jnp.bfloat16),
        b2.reshape(1, hid),
    )


# speedup vs baseline: 1.0031x; 1.0009x over previous
import jax
import jax.numpy as jnp
from jax.experimental import pallas as pl
from jax.experimental.pallas import tpu as pltpu

_CHUNK = 4000
_NBUF = 8


def _mlp_body(x_hbm, w1_ref, b1_ref, w2_ref, b2_ref, o_hbm,
              x_buf, o_buf, in_sems, out_sems):
    n = x_hbm.shape[0]
    nchunks = n // _CHUNK

    def in_copy(i, slot):
        return pltpu.make_async_copy(
            x_hbm.at[pl.ds(i * _CHUNK, _CHUNK), :],
            x_buf.at[slot],
            in_sems.at[slot],
        )

    def out_copy(i, slot):
        return pltpu.make_async_copy(
            o_buf.at[slot],
            o_hbm.at[pl.ds(i * _CHUNK, _CHUNK), :],
            out_sems.at[slot],
        )

    for k in range(_NBUF):
        in_copy(k, k).start()

    def mlp(xblk):
        h = jnp.tanh(
            jnp.dot(
                xblk.astype(jnp.bfloat16),
                w1_ref[...],
                preferred_element_type=jnp.float32,
            )
            + b1_ref[...]
        )
        return jnp.tanh(
            jnp.dot(
                h.astype(jnp.bfloat16),
                w2_ref[...],
                preferred_element_type=jnp.float32,
            )
            + b2_ref[...]
        )

    def step(i, carry):
        slot = jax.lax.rem(i, _NBUF)

        @pl.when(i >= _NBUF)
        def _():
            out_copy(i - _NBUF, slot).wait()

        in_copy(i, slot).wait()
        o_buf[slot] = mlp(x_buf[slot])
        out_copy(i, slot).start()

        @pl.when(i + _NBUF < nchunks)
        def _():
            in_copy(i + _NBUF, slot).start()

        return carry

    jax.lax.fori_loop(0, nchunks, step, 0)

    for k in range(nchunks - _NBUF, nchunks):
        out_copy(k, k % _NBUF).wait()


def kernel(x, W1, b1, W2, b2):
    n, hid = x.shape
    return pl.pallas_call(
        _mlp_body,
        in_specs=[
            pl.BlockSpec(memory_space=pl.ANY),
            pl.BlockSpec(memory_space=pltpu.MemorySpace.VMEM),
            pl.BlockSpec(memory_space=pltpu.MemorySpace.VMEM),
            pl.BlockSpec(memory_space=pltpu.MemorySpace.VMEM),
            pl.BlockSpec(memory_space=pltpu.MemorySpace.VMEM),
        ],
        out_specs=pl.BlockSpec(memory_space=pl.ANY),
        out_shape=jax.ShapeDtypeStruct((n, hid), jnp.float32),
        scratch_shapes=[
            pltpu.VMEM((_NBUF, _CHUNK, hid), jnp.float32),
            pltpu.VMEM((_NBUF, _CHUNK, hid), jnp.float32),
            pltpu.SemaphoreType.DMA((_NBUF,)),
            pltpu.SemaphoreType.DMA((_NBUF,)),
        ],
        compiler_params=pltpu.CompilerParams(
            disable_semaphore_checks=True,
            skip_device_barrier=True,
        ),
    )(
        x,
        W1.T.astype(jnp.bfloat16),
        b1.reshape(1, hid),
        W2.T.astype(jnp.bfloat16),
        b2.reshape(1, hid),
    )


# speedup vs baseline: 1.0150x; 1.0119x over previous
import jax
import jax.numpy as jnp
from jax.experimental import pallas as pl
from jax.experimental.pallas import tpu as pltpu

_CHUNK = 4000
_NBUF = 8


def _mlp_body(x_hbm, w1_ref, b1_ref, w2_ref, b2_ref, o_hbm,
              x_buf, o_buf, in_sems, out_sems):
    n = x_hbm.shape[0]
    nchunks = n // _CHUNK

    def in_copy(i, slot):
        return pltpu.make_async_copy(
            x_hbm.at[pl.ds(i * _CHUNK, _CHUNK), :],
            x_buf.at[slot],
            in_sems.at[slot],
        )

    def out_copy(i, slot):
        return pltpu.make_async_copy(
            o_buf.at[slot],
            o_hbm.at[pl.ds(i * _CHUNK, _CHUNK), :],
            out_sems.at[slot],
        )

    for k in range(_NBUF):
        in_copy(k, k).start()

    def mlp(xblk):
        h = jnp.tanh(
            jnp.dot(
                xblk.astype(jnp.bfloat16),
                w1_ref[...],
                preferred_element_type=jnp.float32,
            )
            + b1_ref[...]
        )
        return jnp.tanh(
            jnp.dot(
                h.astype(jnp.bfloat16),
                w2_ref[...],
                preferred_element_type=jnp.float32,
            )
            + b2_ref[...]
        )

    def step(i, carry):
        slot = jax.lax.rem(i, _NBUF)

        @pl.when(i >= _NBUF)
        def _():
            out_copy(i - _NBUF, slot).wait()

        in_copy(i, slot).wait()
        o_buf[slot] = mlp(x_buf[slot])
        out_copy(i, slot).start()

        @pl.when(i + _NBUF < nchunks)
        def _():
            in_copy(i + _NBUF, slot).start()

        return carry

    jax.lax.fori_loop(0, nchunks - 1, step, 0)

    last = nchunks - 1
    lslot = last % _NBUF
    half = _CHUNK // 2
    out_copy(last - _NBUF, lslot).wait()
    in_copy(last, lslot).wait()
    z = mlp(x_buf[lslot])
    last_halves = [
        pltpu.make_async_copy(
            o_buf.at[lslot, pl.ds(q * half, half), :],
            o_hbm.at[pl.ds(last * _CHUNK + q * half, half), :],
            out_sems.at[lslot],
        )
        for q in range(2)
    ]
    for q in range(2):
        o_buf[lslot, pl.ds(q * half, half), :] = z[q * half:(q + 1) * half, :]
        last_halves[q].start()

    for k in range(nchunks - _NBUF, nchunks - 1):
        out_copy(k, k % _NBUF).wait()
    for q in range(2):
        last_halves[q].wait()


def kernel(x, W1, b1, W2, b2):
    n, hid = x.shape
    return pl.pallas_call(
        _mlp_body,
        in_specs=[
            pl.BlockSpec(memory_space=pl.ANY),
            pl.BlockSpec(memory_space=pltpu.MemorySpace.VMEM),
            pl.BlockSpec(memory_space=pltpu.MemorySpace.VMEM),
            pl.BlockSpec(memory_space=pltpu.MemorySpace.VMEM),
            pl.BlockSpec(memory_space=pltpu.MemorySpace.VMEM),
        ],
        out_specs=pl.BlockSpec(memory_space=pl.ANY),
        out_shape=jax.ShapeDtypeStruct((n, hid), jnp.float32),
        scratch_shapes=[
            pltpu.VMEM((_NBUF, _CHUNK, hid), jnp.float32),
            pltpu.VMEM((_NBUF, _CHUNK, hid), jnp.float32),
            pltpu.SemaphoreType.DMA((_NBUF,)),
            pltpu.SemaphoreType.DMA((_NBUF,)),
        ],
        compiler_params=pltpu.CompilerParams(
            disable_semaphore_checks=True,
            skip_device_barrier=True,
        ),
    )(
        x,
        W1.T.astype(jnp.bfloat16),
        b1.reshape(1, hid),
        W2.T.astype(jnp.bfloat16),
        b2.reshape(1, hid),
    )
